# Initial kernel scaffold; baseline (speedup 1.0000x reference)
import jax
import jax.numpy as jnp
from jax.experimental import pallas as pl


def kernel(token_indices, token_embedding_table):
    raise NotImplementedError("write your pallas kernel here")


# speedup vs baseline: 1.0071x; 1.0071x over previous
import functools

import jax
import jax.numpy as jnp
from jax import lax
from jax.experimental import pallas as pl
from jax.experimental.pallas import tpu as pltpu
from jax.experimental.pallas import tpu_sc as plsc

VOCAB = 1000
NUM_CORES = 2
NUM_SUBCORES = 16
NUM_WORKERS = NUM_CORES * NUM_SUBCORES
CHUNK = 64


def _make_gather(batch: int, dim: int):
    assert batch % (8 * NUM_WORKERS) == 0
    per_w = batch // NUM_WORKERS
    assert per_w % CHUNK == 0
    n_chunks = per_w // CHUNK

    mesh = plsc.VectorSubcoreMesh(core_axis_name="c", subcore_axis_name="s")

    @functools.partial(
        pl.kernel,
        mesh=mesh,
        compiler_params=pltpu.CompilerParams(use_tc_tiling_on_sc=False),
        out_type=jax.ShapeDtypeStruct((batch, dim), jnp.float32),
        scratch_types=[
            pltpu.VMEM((CHUNK,), jnp.int32),
            pltpu.VMEM((CHUNK, dim), jnp.float32),
            pltpu.SemaphoreType.DMA,
        ],
    )
    def gather_kernel(table_hbm, idx_hbm, out_hbm, idx_v, rows_v, sem):
        wid = lax.axis_index("s") * NUM_CORES + lax.axis_index("c")
        base_w = wid * per_w

        def body(c, carry):
            base = base_w + c * CHUNK
            pltpu.sync_copy(idx_hbm.at[pl.ds(base, CHUNK)], idx_v)
            pltpu.async_copy(table_hbm.at[idx_v], rows_v, sem).wait()
            pltpu.sync_copy(rows_v, out_hbm.at[pl.ds(base, CHUNK), :])
            return carry

        lax.fori_loop(0, n_chunks, body, 0)

    return gather_kernel


def kernel(token_indices, token_embedding_table):
    b, t = token_indices.shape
    v, d = token_embedding_table.shape
    idx_flat = token_indices.reshape(b * t).astype(jnp.int32)
    out = _make_gather(b * t, d)(token_embedding_table, idx_flat)
    return out.reshape(b, t, d)


# speedup vs baseline: 1.0273x; 1.0201x over previous
import functools

import jax
import jax.numpy as jnp
from jax import lax
from jax.experimental import pallas as pl
from jax.experimental.pallas import tpu as pltpu
from jax.experimental.pallas import tpu_sc as plsc

NUM_CORES = 2
NUM_SUBCORES = 16
NUM_WORKERS = NUM_CORES * NUM_SUBCORES
CHUNK = 40


def _make_gather(batch: int, dim: int):
    assert batch % (8 * NUM_WORKERS) == 0
    per_w = batch // NUM_WORKERS
    assert per_w % (2 * CHUNK) == 0
    n_chunks = per_w // CHUNK

    mesh = plsc.VectorSubcoreMesh(core_axis_name="c", subcore_axis_name="s")

    @functools.partial(
        pl.kernel,
        mesh=mesh,
        compiler_params=pltpu.CompilerParams(use_tc_tiling_on_sc=False),
        out_type=jax.ShapeDtypeStruct((batch, dim), jnp.float32),
        scratch_types=[
            pltpu.VMEM((per_w,), jnp.int32),
            pltpu.VMEM((2, CHUNK, dim), jnp.float32),
            pltpu.SemaphoreType.DMA,
            pltpu.SemaphoreType.DMA,
            pltpu.SemaphoreType.DMA,
            pltpu.SemaphoreType.DMA,
        ],
    )
    def gather_kernel(table_hbm, idx_hbm, out_hbm, idx_all, rows_v, g0, g1, s0, s1):
        wid = lax.axis_index("s") * NUM_CORES + lax.axis_index("c")
        base_w = wid * per_w
        pltpu.sync_copy(idx_hbm.at[pl.ds(base_w, per_w)], idx_all)
        gsem = (g0, g1)
        ssem = (s0, s1)

        def g_start(c, b):
            pltpu.async_copy(
                table_hbm.at[idx_all.at[pl.ds(c * CHUNK, CHUNK)]],
                rows_v.at[b], gsem[b])

        def g_wait(b):
            pltpu.make_async_copy(
                table_hbm.at[pl.ds(0, CHUNK), :], rows_v.at[b], gsem[b]).wait()

        def s_start(c, b):
            pltpu.async_copy(
                rows_v.at[b],
                out_hbm.at[pl.ds(base_w + c * CHUNK, CHUNK), :], ssem[b])

        def s_wait(b):
            pltpu.make_async_copy(
                rows_v.at[b], out_hbm.at[pl.ds(base_w, CHUNK), :], ssem[b]).wait()

        def step(c, b):
            g_wait(b)
            s_start(c, b)
            s_wait(1 - b)
            g_start(c + 1, 1 - b)

        g_start(0, 0)
        g_wait(0)
        s_start(0, 0)
        g_start(1, 1)

        @pl.loop(1, n_chunks - 1, step=2)
        def _(c):
            step(c, 1)
            step(c + 1, 0)

        g_wait(1)
        s_start(n_chunks - 1, 1)
        s_wait(0)
        s_wait(1)

    return gather_kernel


def kernel(token_indices, token_embedding_table):
    b, t = token_indices.shape
    v, d = token_embedding_table.shape
    idx_flat = token_indices.reshape(b * t).astype(jnp.int32)
    out = _make_gather(b * t, d)(token_embedding_table, idx_flat)
    return out.reshape(b, t, d)
